# baseline (speedup 1.0000x reference)
import jax
import jax.numpy as jnp
from jax.experimental import pallas as pl
from jax.experimental.pallas import tpu as pltpu

SCORE_T = 0.5
NMS_T = 0.5


def _nms_body(b_ref, sc_ref, sr_ref, lab_ref,
              bx_ref, lb_ref, so_ref, kp_ref, kb_ref, pm_ref, src_ref,
              iou_ref):
    n = b_ref.shape[0]
    i0 = jax.lax.broadcasted_iota(jnp.int32, (n, n), 0)
    i1 = jax.lax.broadcasted_iota(jnp.int32, (n, n), 1)
    sa = sc_ref[...]
    sb = sr_ref[...]

    cmp_ij = (sb > sa) | ((sb == sa) & (i1 < i0))
    rank_c = jnp.sum(cmp_ij.astype(jnp.int32), axis=1, keepdims=True)
    cmp_ji = (sa > sb) | ((sa == sb) & (i0 < i1))
    rank_r = jnp.sum(cmp_ji.astype(jnp.int32), axis=0, keepdims=True)

    eq = rank_r == i0
    eqT = rank_c == i1
    eqTf = eqT.astype(jnp.float32)

    diag = i0 == i1

    def grow(col_vals):
        return jnp.sum(eqTf * col_vals, axis=0, keepdims=True)

    def to_col(row_vals):
        return jnp.sum(jnp.where(diag, row_vals, 0.0), axis=1, keepdims=True)

    x1_r = grow(b_ref[:, 0:1]); y1_r = grow(b_ref[:, 1:2])
    x2_r = grow(b_ref[:, 2:3]); y2_r = grow(b_ref[:, 3:4])
    x1_c = to_col(x1_r); y1_c = to_col(y1_r)
    x2_c = to_col(x2_r); y2_c = to_col(y2_r)
    s_r = grow(sa)
    s_c = to_col(s_r)
    lab_c = jnp.sum(eq.astype(jnp.int32) * lab_ref[...], axis=1, keepdims=True)
    perm_c = jnp.sum(eq.astype(jnp.int32) * i1, axis=1, keepdims=True)
    perm_r = jnp.sum(eqT.astype(jnp.int32) * i0, axis=0, keepdims=True)

    xx1 = jnp.maximum(x1_c, x1_r)
    yy1 = jnp.maximum(y1_c, y1_r)
    xx2 = jnp.minimum(x2_c, x2_r)
    yy2 = jnp.minimum(y2_c, y2_r)
    inter = jnp.maximum(xx2 - xx1, 0.0) * jnp.maximum(yy2 - yy1, 0.0)
    area_c = (x2_c - x1_c) * (y2_c - y1_c)
    area_r = (x2_r - x1_r) * (y2_r - y1_r)
    iou_ref[...] = inter / (area_c + area_r - inter)

    valid = s_r > SCORE_T
    kcount = jnp.sum(valid.astype(jnp.int32))
    nchunks = (kcount + 7) // 8
    idxr = jax.lax.broadcasted_iota(jnp.int32, (1, n), 1)

    def chunk_body(c, keep):
        base = pl.multiple_of(c * 8, 8)
        chunk = iou_ref[pl.ds(base, 8), :]
        for t in range(8):
            i = c * 8 + t
            row = chunk[t:t + 1, :]
            sup = jnp.any((idxr < i) & (keep != 0) & (row > NMS_T))
            keep = jnp.where((idxr == i) & sup, 0, keep)
        return keep

    keep_i = jax.lax.fori_loop(0, nchunks, chunk_body,
                               valid.astype(jnp.int32))
    keep = keep_i != 0

    keep_ci = jnp.sum(((i0 == i1) & keep).astype(jnp.int32),
                      axis=1, keepdims=True)
    keep_cf = keep_ci.astype(jnp.float32)

    bx_ref[:, 0:1] = x1_c * keep_cf
    bx_ref[:, 1:2] = y1_c * keep_cf
    bx_ref[:, 2:3] = x2_c * keep_cf
    bx_ref[:, 3:4] = y2_c * keep_cf
    lb_ref[...] = lab_c * keep_ci
    so_ref[...] = s_c * keep_cf
    kp_ref[...] = keep_ci
    kb_ref[...] = keep_ci != 0
    pm_ref[...] = perm_c

    g = _ROWS_PER_STEP
    lane_ok = (i1 <= i0) & (((i0 - i1) % g) == 0) & keep
    t_col = jnp.max(jnp.where(lane_ok, i1, -1),
                    axis=1, keepdims=True)
    sel = (i1 == t_col).astype(jnp.int32)
    src_col = jnp.sum(sel * perm_r, axis=1, keepdims=True)
    src_ref[...] = jnp.where(t_col < 0, perm_c, src_col)


_ROWS_PER_STEP = 8


def _mask_body(src_ref, keep_ref, *refs):
    g = _ROWS_PER_STEP
    o_ref = refs[g]
    r = pl.program_id(0)
    for j in range(g):
        k = keep_ref[r * g + j].astype(jnp.float32)
        o_ref[j] = refs[j][0] * k


def kernel(boxes, scores, labels, masks):
    n = boxes.shape[0]
    h, w = masks.shape[2], masks.shape[3]

    bx, lb, so, kp, kb, pm, src = pl.pallas_call(
        _nms_body,
        out_shape=[
            jax.ShapeDtypeStruct((n, 4), jnp.float32),
            jax.ShapeDtypeStruct((n, 1), jnp.int32),
            jax.ShapeDtypeStruct((n, 1), jnp.float32),
            jax.ShapeDtypeStruct((n, 1), jnp.int32),
            jax.ShapeDtypeStruct((n, 1), jnp.bool_),
            jax.ShapeDtypeStruct((n, 1), jnp.int32),
            jax.ShapeDtypeStruct((n, 1), jnp.int32),
        ],
        scratch_shapes=[pltpu.VMEM((n, n), jnp.float32)],
        name="nms_sort",
    )(boxes, scores[:, None], scores[None, :], labels[None, :])

    keep_i = kp[:, 0]
    g = _ROWS_PER_STEP
    m3 = masks.reshape(n, h, w)

    def make_in_spec(j):
        return pl.BlockSpec((1, h, w),
                            lambda r, src, keep, j=j: (src[r * g + j], 0, 0))

    masks_out = pl.pallas_call(
        _mask_body,
        grid_spec=pltpu.PrefetchScalarGridSpec(
            num_scalar_prefetch=2,
            grid=(n // g,),
            in_specs=[make_in_spec(j) for j in range(g)],
            out_specs=pl.BlockSpec((g, h, w),
                                   lambda r, src, keep: (r, 0, 0)),
        ),
        out_shape=jax.ShapeDtypeStruct((n, h, w), jnp.float32),
        compiler_params=pltpu.CompilerParams(
            dimension_semantics=("arbitrary",)),
        name="mask_gather",
    )(src[:, 0], keep_i, *([m3] * g))

    return (bx, lb[:, 0], so[:, 0], masks_out.reshape(masks.shape),
            kb[:, 0])


# speedup vs baseline: 1.0661x; 1.0661x over previous
import jax
import jax.numpy as jnp
from jax.experimental import pallas as pl
from jax.experimental.pallas import tpu as pltpu

SCORE_T = 0.5
NMS_T = 0.5


def _nms_body(b_ref, sc_ref, sr_ref, lab_ref,
              bx_ref, lb_ref, so_ref, kp_ref, kb_ref, pm_ref, src_ref,
              iou_ref, blk_ref):
    n = b_ref.shape[0]
    i0 = jax.lax.broadcasted_iota(jnp.int32, (n, n), 0)
    i1 = jax.lax.broadcasted_iota(jnp.int32, (n, n), 1)
    sa = sc_ref[...]
    sb = sr_ref[...]

    cmp_ij = (sb > sa) | ((sb == sa) & (i1 < i0))
    rank_c = jnp.sum(cmp_ij.astype(jnp.int32), axis=1, keepdims=True)
    cmp_ji = (sa > sb) | ((sa == sb) & (i0 < i1))
    rank_r = jnp.sum(cmp_ji.astype(jnp.int32), axis=0, keepdims=True)

    eq = rank_r == i0
    eqT = rank_c == i1
    eqTf = eqT.astype(jnp.float32)

    diag = i0 == i1

    def grow(col_vals):
        return jnp.sum(eqTf * col_vals, axis=0, keepdims=True)

    def to_col(row_vals):
        return jnp.sum(jnp.where(diag, row_vals, 0.0), axis=1, keepdims=True)

    x1_r = grow(b_ref[:, 0:1]); y1_r = grow(b_ref[:, 1:2])
    x2_r = grow(b_ref[:, 2:3]); y2_r = grow(b_ref[:, 3:4])
    x1_c = to_col(x1_r); y1_c = to_col(y1_r)
    x2_c = to_col(x2_r); y2_c = to_col(y2_r)
    s_r = grow(sa)
    s_c = to_col(s_r)
    lab_c = jnp.sum(eq.astype(jnp.int32) * lab_ref[...], axis=1, keepdims=True)
    perm_c = jnp.sum(eq.astype(jnp.int32) * i1, axis=1, keepdims=True)
    perm_r = jnp.sum(eqT.astype(jnp.int32) * i0, axis=0, keepdims=True)

    xx1 = jnp.maximum(x1_c, x1_r)
    yy1 = jnp.maximum(y1_c, y1_r)
    xx2 = jnp.minimum(x2_c, x2_r)
    yy2 = jnp.minimum(y2_c, y2_r)
    inter = jnp.maximum(xx2 - xx1, 0.0) * jnp.maximum(yy2 - yy1, 0.0)
    area_c = (x2_c - x1_c) * (y2_c - y1_c)
    area_r = (x2_r - x1_r) * (y2_r - y1_r)
    iou_ref[...] = inter / (area_c + area_r - inter)

    valid = s_r > SCORE_T
    kcount = jnp.sum(valid.astype(jnp.int32))
    nchunks = (kcount + 7) // 8
    idxr = jax.lax.broadcasted_iota(jnp.int32, (1, n), 1)
    lane8 = jax.lax.broadcasted_iota(jnp.int32, (1, 8), 1)
    blk_ref[...] = jnp.zeros_like(blk_ref)

    def chunk_body(c, keep):
        base = pl.multiple_of(c * 8, 8)
        chunk_b = (iou_ref[pl.ds(base, 8), :] > NMS_T).astype(jnp.int32)
        ext8 = pltpu.roll(blk_ref[...], -base, axis=1)[0:1, 0:8]
        a8 = pltpu.roll(chunk_b, -base, axis=1)[:, 0:8]
        valid8 = ((base + lane8) < kcount).astype(jnp.int32)
        kept8 = valid8 * (1 - ext8)
        for u in range(7):
            ku = kept8[0, u]
            kept8 = kept8 * (1 - a8[u:u + 1, :] * ku
                             * (lane8 > u).astype(jnp.int32))
        contrib = chunk_b[0:1, :] * kept8[0, 0]
        for u in range(1, 8):
            contrib = jnp.maximum(contrib, chunk_b[u:u + 1, :] * kept8[0, u])
        blk_ref[...] = jnp.maximum(blk_ref[...], contrib)
        placed = pltpu.roll(
            jnp.concatenate([kept8, jnp.zeros((1, n - 8), jnp.int32)],
                            axis=1), base, axis=1)
        in_chunk = (idxr >= base) & (idxr < base + 8)
        return jnp.where(in_chunk, placed, keep)

    keep_i = jax.lax.fori_loop(0, nchunks, chunk_body,
                               valid.astype(jnp.int32))
    keep = keep_i != 0

    keep_ci = jnp.sum(((i0 == i1) & keep).astype(jnp.int32),
                      axis=1, keepdims=True)
    keep_cf = keep_ci.astype(jnp.float32)

    bx_ref[:, 0:1] = x1_c * keep_cf
    bx_ref[:, 1:2] = y1_c * keep_cf
    bx_ref[:, 2:3] = x2_c * keep_cf
    bx_ref[:, 3:4] = y2_c * keep_cf
    lb_ref[...] = lab_c * keep_ci
    so_ref[...] = s_c * keep_cf
    kp_ref[...] = keep_ci
    kb_ref[...] = keep_ci != 0
    pm_ref[...] = perm_c

    g = _ROWS_PER_STEP
    lane_ok = (i1 <= i0) & (((i0 - i1) % g) == 0) & keep
    t_col = jnp.max(jnp.where(lane_ok, i1, -1),
                    axis=1, keepdims=True)
    sel = (i1 == t_col).astype(jnp.int32)
    src_col = jnp.sum(sel * perm_r, axis=1, keepdims=True)
    src_ref[...] = jnp.where(t_col < 0, perm_c, src_col)


_ROWS_PER_STEP = 8


def _mask_body(src_ref, keep_ref, *refs):
    g = _ROWS_PER_STEP
    o_ref = refs[g]
    r = pl.program_id(0)
    for j in range(g):
        k = keep_ref[r * g + j].astype(jnp.float32)
        o_ref[j] = refs[j][0] * k


def kernel(boxes, scores, labels, masks):
    n = boxes.shape[0]
    h, w = masks.shape[2], masks.shape[3]

    bx, lb, so, kp, kb, pm, src = pl.pallas_call(
        _nms_body,
        out_shape=[
            jax.ShapeDtypeStruct((n, 4), jnp.float32),
            jax.ShapeDtypeStruct((n, 1), jnp.int32),
            jax.ShapeDtypeStruct((n, 1), jnp.float32),
            jax.ShapeDtypeStruct((n, 1), jnp.int32),
            jax.ShapeDtypeStruct((n, 1), jnp.bool_),
            jax.ShapeDtypeStruct((n, 1), jnp.int32),
            jax.ShapeDtypeStruct((n, 1), jnp.int32),
        ],
        scratch_shapes=[pltpu.VMEM((n, n), jnp.float32),
                        pltpu.VMEM((1, n), jnp.int32)],
        name="nms_sort",
    )(boxes, scores[:, None], scores[None, :], labels[None, :])

    keep_i = kp[:, 0]
    g = _ROWS_PER_STEP
    m3 = masks.reshape(n, h, w)

    def make_in_spec(j):
        return pl.BlockSpec((1, h, w),
                            lambda r, src, keep, j=j: (src[r * g + j], 0, 0))

    masks_out = pl.pallas_call(
        _mask_body,
        grid_spec=pltpu.PrefetchScalarGridSpec(
            num_scalar_prefetch=2,
            grid=(n // g,),
            in_specs=[make_in_spec(j) for j in range(g)],
            out_specs=pl.BlockSpec((g, h, w),
                                   lambda r, src, keep: (r, 0, 0)),
        ),
        out_shape=jax.ShapeDtypeStruct((n, h, w), jnp.float32),
        compiler_params=pltpu.CompilerParams(
            dimension_semantics=("arbitrary",)),
        name="mask_gather",
    )(src[:, 0], keep_i, *([m3] * g))

    return (bx, lb[:, 0], so[:, 0], masks_out.reshape(masks.shape),
            kb[:, 0])


# speedup vs baseline: 1.1184x; 1.0490x over previous
import jax
import jax.numpy as jnp
from jax.experimental import pallas as pl
from jax.experimental.pallas import tpu as pltpu

SCORE_T = 0.5
NMS_T = 0.5


def _nms_body(b_ref, sc_ref, sr_ref, lab_ref,
              bx_ref, lb_ref, so_ref, kp_ref, kb_ref, pm_ref, src_ref,
              iou_ref, blk_ref):
    n = b_ref.shape[0]
    i0 = jax.lax.broadcasted_iota(jnp.int32, (n, n), 0)
    i1 = jax.lax.broadcasted_iota(jnp.int32, (n, n), 1)
    sa = sc_ref[...]
    sb = sr_ref[...]

    cmp_ij = (sb > sa) | ((sb == sa) & (i1 < i0))
    rank_c = jnp.sum(cmp_ij.astype(jnp.int32), axis=1, keepdims=True)
    cmp_ji = (sa > sb) | ((sa == sb) & (i0 < i1))
    rank_r = jnp.sum(cmp_ji.astype(jnp.int32), axis=0, keepdims=True)

    eq = rank_r == i0
    eqT = rank_c == i1
    eqTf = eqT.astype(jnp.float32)

    diag = i0 == i1

    def grow(col_vals):
        return jnp.sum(eqTf * col_vals, axis=0, keepdims=True)

    def to_col(row_vals):
        return jnp.sum(jnp.where(diag, row_vals, 0.0), axis=1, keepdims=True)

    x1_r = grow(b_ref[:, 0:1]); y1_r = grow(b_ref[:, 1:2])
    x2_r = grow(b_ref[:, 2:3]); y2_r = grow(b_ref[:, 3:4])
    x1_c = to_col(x1_r); y1_c = to_col(y1_r)
    x2_c = to_col(x2_r); y2_c = to_col(y2_r)
    s_r = grow(sa)
    s_c = to_col(s_r)
    lab_c = jnp.sum(eq.astype(jnp.int32) * lab_ref[...], axis=1, keepdims=True)
    perm_c = jnp.sum(eq.astype(jnp.int32) * i1, axis=1, keepdims=True)
    perm_r = jnp.sum(eqT.astype(jnp.int32) * i0, axis=0, keepdims=True)

    xx1 = jnp.maximum(x1_c, x1_r)
    yy1 = jnp.maximum(y1_c, y1_r)
    xx2 = jnp.minimum(x2_c, x2_r)
    yy2 = jnp.minimum(y2_c, y2_r)
    inter = jnp.maximum(xx2 - xx1, 0.0) * jnp.maximum(yy2 - yy1, 0.0)
    area_c = (x2_c - x1_c) * (y2_c - y1_c)
    area_r = (x2_r - x1_r) * (y2_r - y1_r)
    iou_ref[...] = inter / (area_c + area_r - inter)

    valid = s_r > SCORE_T
    kcount = jnp.sum(valid.astype(jnp.int32))
    nchunks = (kcount + 7) // 8
    idxr = jax.lax.broadcasted_iota(jnp.int32, (1, n), 1)
    lane8 = jax.lax.broadcasted_iota(jnp.int32, (1, 8), 1)
    blk_ref[...] = jnp.zeros_like(blk_ref)

    lanepow = jnp.left_shift(jnp.int32(1), lane8)

    def chunk_body(c, keep):
        base = pl.multiple_of(c * 8, 8)
        chunk_b = (iou_ref[pl.ds(base, 8), :] > NMS_T).astype(jnp.int32)
        ext8 = pltpu.roll(blk_ref[...], -base, axis=1)[0:1, 0:8]
        a8 = pltpu.roll(chunk_b, -base, axis=1)[:, 0:8]
        valid8 = ((base + lane8) < kcount).astype(jnp.int32)
        a8m = jnp.sum(a8 * lanepow, axis=1, keepdims=True)
        kinit = jnp.sum(valid8 * (1 - ext8) * lanepow)
        kmask = kinit
        for u in range(7):
            arow = a8m[u, 0]
            bit = jax.lax.shift_right_logical(kmask, u) & 1
            kill = arow * bit & (-2 << u)
            kmask = kmask & ~kill
        kept8 = jax.lax.shift_right_logical(
            jnp.broadcast_to(kmask, (1, 8)), lane8) & 1
        contrib = chunk_b[0:1, :] * (kmask & 1)
        for u in range(1, 8):
            contrib = jnp.maximum(
                contrib,
                chunk_b[u:u + 1, :] * (jax.lax.shift_right_logical(kmask, u)
                                       & 1))
        blk_ref[...] = jnp.maximum(blk_ref[...], contrib)
        placed = pltpu.roll(
            jnp.concatenate([kept8, jnp.zeros((1, n - 8), jnp.int32)],
                            axis=1), base, axis=1)
        in_chunk = (idxr >= base) & (idxr < base + 8)
        return jnp.where(in_chunk, placed, keep)

    keep_i = jax.lax.fori_loop(0, nchunks, chunk_body,
                               valid.astype(jnp.int32))
    keep = keep_i != 0

    keep_ci = jnp.sum(((i0 == i1) & keep).astype(jnp.int32),
                      axis=1, keepdims=True)
    keep_cf = keep_ci.astype(jnp.float32)

    bx_ref[:, 0:1] = x1_c * keep_cf
    bx_ref[:, 1:2] = y1_c * keep_cf
    bx_ref[:, 2:3] = x2_c * keep_cf
    bx_ref[:, 3:4] = y2_c * keep_cf
    lb_ref[...] = lab_c * keep_ci
    so_ref[...] = s_c * keep_cf
    kp_ref[...] = keep_ci
    kb_ref[...] = keep_ci != 0
    pm_ref[...] = perm_c

    g = _ROWS_PER_STEP
    lane_ok = (i1 <= i0) & (((i0 - i1) % g) == 0) & keep
    t_col = jnp.max(jnp.where(lane_ok, i1, -1),
                    axis=1, keepdims=True)
    sel = (i1 == t_col).astype(jnp.int32)
    src_col = jnp.sum(sel * perm_r, axis=1, keepdims=True)
    src_ref[...] = jnp.where(t_col < 0, perm_c, src_col)


_ROWS_PER_STEP = 8


def _mask_body(src_ref, keep_ref, *refs):
    g = _ROWS_PER_STEP
    o_ref = refs[g]
    r = pl.program_id(0)
    for j in range(g):
        k = keep_ref[r * g + j].astype(jnp.float32)
        o_ref[j] = refs[j][0] * k


def kernel(boxes, scores, labels, masks):
    n = boxes.shape[0]
    h, w = masks.shape[2], masks.shape[3]

    bx, lb, so, kp, kb, pm, src = pl.pallas_call(
        _nms_body,
        out_shape=[
            jax.ShapeDtypeStruct((n, 4), jnp.float32),
            jax.ShapeDtypeStruct((n, 1), jnp.int32),
            jax.ShapeDtypeStruct((n, 1), jnp.float32),
            jax.ShapeDtypeStruct((n, 1), jnp.int32),
            jax.ShapeDtypeStruct((n, 1), jnp.bool_),
            jax.ShapeDtypeStruct((n, 1), jnp.int32),
            jax.ShapeDtypeStruct((n, 1), jnp.int32),
        ],
        scratch_shapes=[pltpu.VMEM((n, n), jnp.float32),
                        pltpu.VMEM((1, n), jnp.int32)],
        name="nms_sort",
    )(boxes, scores[:, None], scores[None, :], labels[None, :])

    keep_i = kp[:, 0]
    g = _ROWS_PER_STEP
    m3 = masks.reshape(n, h, w)

    def make_in_spec(j):
        return pl.BlockSpec((1, h, w),
                            lambda r, src, keep, j=j: (src[r * g + j], 0, 0))

    masks_out = pl.pallas_call(
        _mask_body,
        grid_spec=pltpu.PrefetchScalarGridSpec(
            num_scalar_prefetch=2,
            grid=(n // g,),
            in_specs=[make_in_spec(j) for j in range(g)],
            out_specs=pl.BlockSpec((g, h, w),
                                   lambda r, src, keep: (r, 0, 0)),
        ),
        out_shape=jax.ShapeDtypeStruct((n, h, w), jnp.float32),
        compiler_params=pltpu.CompilerParams(
            dimension_semantics=("arbitrary",)),
        name="mask_gather",
    )(src[:, 0], keep_i, *([m3] * g))

    return (bx, lb[:, 0], so[:, 0], masks_out.reshape(masks.shape),
            kb[:, 0])
